# Initial kernel scaffold; baseline (speedup 1.0000x reference)
import jax
import jax.numpy as jnp
from jax.experimental import pallas as pl


def kernel(h, pos, edge_index, W_e1, b_e1, W_e2, b_e2, W_n1, b_n1, W_n2, b_n2, W_c1, b_c1, W_c2, ln_gamma, ln_beta):
    raise NotImplementedError("write your pallas kernel here")


# speedup vs baseline: 3.1180x; 3.1180x over previous
import functools

import jax
import jax.numpy as jnp
from jax import lax
from jax.experimental import pallas as pl
from jax.experimental.pallas import tpu as pltpu
from jax.experimental.pallas import tpu_sc as plsc

HID = 128
PPAD = 16
NC, NS = 2, 16
NW = NC * NS
C = 80


def _silu(x):
  return x * jax.nn.sigmoid(x)


def _sc_gather_body(epw, nchunk, h_hbm, posp_hbm, row_hbm, col_hbm,
                    hr_hbm, hc_hbm, pr_hbm, pc_hbm,
                    idx_r, idx_c, hbuf_r, hbuf_c, pbuf_r, pbuf_c, sem):
  wid = lax.axis_index("s") * NC + lax.axis_index("c")
  base = wid * epw

  def chunk(i, carry):
    off = base + i * C
    pltpu.sync_copy(row_hbm.at[pl.ds(off, C)], idx_r)
    pltpu.sync_copy(col_hbm.at[pl.ds(off, C)], idx_c)
    d1 = pltpu.async_copy(h_hbm.at[idx_r], hbuf_r, sem)
    d2 = pltpu.async_copy(h_hbm.at[idx_c], hbuf_c, sem)
    d3 = pltpu.async_copy(posp_hbm.at[idx_r], pbuf_r, sem)
    d4 = pltpu.async_copy(posp_hbm.at[idx_c], pbuf_c, sem)
    d1.wait()
    d2.wait()
    d3.wait()
    d4.wait()
    pltpu.sync_copy(hbuf_r, hr_hbm.at[pl.ds(off, C)])
    pltpu.sync_copy(hbuf_c, hc_hbm.at[pl.ds(off, C)])
    pltpu.sync_copy(pbuf_r, pr_hbm.at[pl.ds(off, C)])
    pltpu.sync_copy(pbuf_c, pc_hbm.at[pl.ds(off, C)])
    return carry

  lax.fori_loop(0, nchunk, chunk, 0)


def _sc_gather(h, posp, row, col):
  n = h.shape[0]
  e = row.shape[0]
  epw = e // NW
  nchunk = epw // C
  mesh = plsc.VectorSubcoreMesh(core_axis_name="c", subcore_axis_name="s",
                                num_cores=NC, num_subcores=NS)
  f = pl.kernel(
      functools.partial(_sc_gather_body, epw, nchunk),
      compiler_params=pltpu.CompilerParams(use_tc_tiling_on_sc=False),
      out_type=(
          jax.ShapeDtypeStruct((e, HID), jnp.float32),
          jax.ShapeDtypeStruct((e, HID), jnp.float32),
          jax.ShapeDtypeStruct((e, PPAD), jnp.float32),
          jax.ShapeDtypeStruct((e, PPAD), jnp.float32),
      ),
      mesh=mesh,
      scratch_types=(
          pltpu.VMEM((C,), jnp.int32),
          pltpu.VMEM((C,), jnp.int32),
          pltpu.VMEM((C, HID), jnp.float32),
          pltpu.VMEM((C, HID), jnp.float32),
          pltpu.VMEM((C, PPAD), jnp.float32),
          pltpu.VMEM((C, PPAD), jnp.float32),
          pltpu.SemaphoreType.DMA,
      ),
  )
  return f(h, posp, row, col)


def _sc_scatter_body(epw, nchunk, row_hbm, m2_hbm, cwd_hbm, zm_hbm, zc_hbm,
                     aggm_hbm, aggc_hbm,
                     idx, mbuf, cbuf, accm, accc, sem):
  cid = lax.axis_index("c")
  sid = lax.axis_index("s")
  wid = sid * NC + cid

  @pl.when(sid == 0)
  def _zero():
    pltpu.sync_copy(zm_hbm, accm)
    pltpu.sync_copy(zc_hbm, accc)

  plsc.subcore_barrier()

  base = wid * epw

  def chunk(i, carry):
    off = base + i * C
    pltpu.sync_copy(row_hbm.at[pl.ds(off, C)], idx)
    pltpu.sync_copy(m2_hbm.at[pl.ds(off, C)], mbuf)
    pltpu.sync_copy(cwd_hbm.at[pl.ds(off, C)], cbuf)
    pltpu.sync_copy(mbuf, accm.at[idx], add=True)
    pltpu.sync_copy(cbuf, accc.at[idx], add=True)
    return carry

  lax.fori_loop(0, nchunk, chunk, 0)

  plsc.subcore_barrier()

  @pl.when(sid == 0)
  def _flush():
    pltpu.sync_copy(accm, aggm_hbm.at[cid])
    pltpu.sync_copy(accc, aggc_hbm.at[cid])


def _sc_scatter(row, m2, cwd, n):
  e = row.shape[0]
  epw = e // NW
  nchunk = epw // C
  zm = jnp.zeros((n, HID), jnp.float32)
  zc = jnp.zeros((n, PPAD), jnp.float32)
  mesh = plsc.VectorSubcoreMesh(core_axis_name="c", subcore_axis_name="s",
                                num_cores=NC, num_subcores=NS)
  f = pl.kernel(
      functools.partial(_sc_scatter_body, epw, nchunk),
      compiler_params=pltpu.CompilerParams(use_tc_tiling_on_sc=False),
      out_type=(
          jax.ShapeDtypeStruct((NC, n, HID), jnp.float32),
          jax.ShapeDtypeStruct((NC, n, PPAD), jnp.float32),
      ),
      mesh=mesh,
      scratch_types=(
          pltpu.VMEM((C,), jnp.int32),
          pltpu.VMEM((C, HID), jnp.float32),
          pltpu.VMEM((C, PPAD), jnp.float32),
          pltpu.VMEM_SHARED((n, HID), jnp.float32),
          pltpu.VMEM_SHARED((n, PPAD), jnp.float32),
          pltpu.SemaphoreType.DMA,
      ),
  )
  return f(row, m2, cwd, zm, zc)


def _tc_edge_body(hr, hc, pr, pc, w1a, w1b, w1d, b1, w2, b2, wc1, bc1, wc2,
                  m2_out, cwd_out):
  diff = pr[...] - pc[...]
  d2 = jnp.sum(diff * diff, axis=1, keepdims=True)
  dist = jnp.sqrt(d2 + 1e-8)
  x = jnp.dot(hr[...], w1a[...], preferred_element_type=jnp.float32)
  x = x + jnp.dot(hc[...], w1b[...], preferred_element_type=jnp.float32)
  x = x + dist * w1d[...] + b1[...]
  m1 = _silu(x)
  y = jnp.dot(m1, w2[...], preferred_element_type=jnp.float32) + b2[...]
  m2 = _silu(y)
  z = _silu(jnp.dot(m2, wc1[...], preferred_element_type=jnp.float32)
            + bc1[...])
  cw = jnp.sum(z * wc2[...], axis=1, keepdims=True)
  m2_out[...] = m2
  cwd_out[...] = (cw / (dist + 1e-8)) * diff


def _tc_edge(hr, hc, pr, pc, w1a, w1b, w1d, b1, w2, b2, wc1, bc1, wc2):
  e = hr.shape[0]
  be = 1280
  grid = (e // be,)
  blk = lambda r, c: pl.BlockSpec((r, c), lambda i: (i, 0))
  wblk = lambda r, c: pl.BlockSpec((r, c), lambda i: (0, 0))
  return pl.pallas_call(
      _tc_edge_body,
      grid=grid,
      in_specs=[
          blk(be, HID), blk(be, HID), blk(be, PPAD), blk(be, PPAD),
          wblk(HID, HID), wblk(HID, HID), wblk(1, HID), wblk(1, HID),
          wblk(HID, HID), wblk(1, HID),
          wblk(HID, HID), wblk(1, HID), wblk(1, HID),
      ],
      out_specs=[blk(be, HID), blk(be, PPAD)],
      out_shape=[
          jax.ShapeDtypeStruct((e, HID), jnp.float32),
          jax.ShapeDtypeStruct((e, PPAD), jnp.float32),
      ],
  )(hr, hc, pr, pc, w1a, w1b, w1d, b1, w2, b2, wc1, bc1, wc2)


def _tc_node_body(h, posp, aggm, aggc, wn1a, wn1b, bn1, wn2, bn2, g, b,
                  h_out, posp_out):
  agg = aggm[0] + aggm[1]
  x = jnp.dot(h[...], wn1a[...], preferred_element_type=jnp.float32)
  x = x + jnp.dot(agg, wn1b[...], preferred_element_type=jnp.float32)
  x = _silu(x + bn1[...])
  hupd = jnp.dot(x, wn2[...], preferred_element_type=jnp.float32) + bn2[...]
  y = h[...] + hupd
  mu = jnp.mean(y, axis=1, keepdims=True)
  var = jnp.mean((y - mu) * (y - mu), axis=1, keepdims=True)
  h_out[...] = (y - mu) * jax.lax.rsqrt(var + 1e-5) * g[...] + b[...]
  posp_out[...] = posp[...] + aggc[0] + aggc[1]


def _tc_node(h, posp, aggm, aggc, wn1a, wn1b, bn1, wn2, bn2, g, b):
  n = h.shape[0]
  bn = 1000
  grid = (n // bn,)
  blk = lambda r, c: pl.BlockSpec((r, c), lambda i: (i, 0))
  wblk = lambda r, c: pl.BlockSpec((r, c), lambda i: (0, 0))
  blk3 = lambda r, c: pl.BlockSpec((NC, r, c), lambda i: (0, i, 0))
  return pl.pallas_call(
      _tc_node_body,
      grid=grid,
      in_specs=[
          blk(bn, HID), blk(bn, PPAD), blk3(bn, HID), blk3(bn, PPAD),
          wblk(HID, HID), wblk(HID, HID), wblk(1, HID),
          wblk(HID, HID), wblk(1, HID), wblk(1, HID), wblk(1, HID),
      ],
      out_specs=[blk(bn, HID), blk(bn, PPAD)],
      out_shape=[
          jax.ShapeDtypeStruct((n, HID), jnp.float32),
          jax.ShapeDtypeStruct((n, PPAD), jnp.float32),
      ],
  )(h, posp, aggm, aggc, wn1a, wn1b, bn1, wn2, bn2, g, b)


@jax.jit
def kernel(h, pos, edge_index, W_e1, b_e1, W_e2, b_e2, W_n1, b_n1, W_n2,
           b_n2, W_c1, b_c1, W_c2, ln_gamma, ln_beta):
  n = h.shape[0]
  row = edge_index[0].astype(jnp.int32)
  col = edge_index[1].astype(jnp.int32)
  posp = jnp.zeros((n, PPAD), jnp.float32).at[:, :3].set(pos)

  hr, hc, pr, pc = _sc_gather(h, posp, row, col)

  w1a = W_e1[:HID]
  w1b = W_e1[HID:2 * HID]
  w1d = W_e1[2 * HID].reshape(1, HID)
  m2, cwd = _tc_edge(hr, hc, pr, pc, w1a, w1b, w1d, b_e1.reshape(1, HID),
                     W_e2, b_e2.reshape(1, HID), W_c1, b_c1.reshape(1, HID),
                     W_c2.reshape(1, HID))

  aggm, aggc = _sc_scatter(row, m2, cwd, n)

  h_out, posp_out = _tc_node(h, posp, aggm, aggc, W_n1[:HID], W_n1[HID:],
                             b_n1.reshape(1, HID), W_n2,
                             b_n2.reshape(1, HID), ln_gamma.reshape(1, HID),
                             ln_beta.reshape(1, HID))
  return h_out, posp_out[:, :3]
